# Initial kernel scaffold; baseline (speedup 1.0000x reference)
import functools

import jax
import jax.numpy as jnp
from jax import lax
from jax.experimental import pallas as pl
from jax.experimental.pallas import tpu as pltpu
from jax.experimental.pallas import tpu_sc as plsc

N = 10000
E = 320000
D = 128

NC = 2
NS = 16
NW = NC * NS

N_PAD = 10240
ROWS_PER_TILE = N_PAD // NS
CHUNK = 128
CHUNKS_PER_TILE = 80
E_PAD = NW * CHUNKS_PER_TILE * CHUNK
DUMMY_DST = N

_mesh = plsc.VectorSubcoreMesh(core_axis_name="c", subcore_axis_name="s")


@functools.partial(
    pl.kernel,
    out_type=jax.ShapeDtypeStruct((NC, N_PAD), jnp.float32),
    mesh=_mesh,
    scratch_types=[
        pltpu.VMEM((CHUNKS_PER_TILE, CHUNK), jnp.int32),
        pltpu.VMEM((CHUNK,), jnp.float32),
        pltpu.VMEM((ROWS_PER_TILE,), jnp.float32),
        pltpu.VMEM_SHARED((N_PAD,), jnp.float32),
    ],
)
def _sc_degree(dstp_hbm, deg_hbm, dst_v, ones_v, zbuf, dacc):
    c = lax.axis_index("c")
    s = lax.axis_index("s")
    wid = s * NC + c

    def fill(i, carry):
        zbuf[pl.ds(i * 16, 16)] = jnp.zeros((16,), jnp.float32)
        ones_v[pl.ds(lax.rem(i, 8) * 16, 16)] = jnp.ones((16,), jnp.float32)
        return carry

    lax.fori_loop(0, ROWS_PER_TILE // 16, fill, 0)
    pltpu.sync_copy(zbuf, dacc.at[pl.ds(s * ROWS_PER_TILE, ROWS_PER_TILE)])
    plsc.subcore_barrier()

    pltpu.sync_copy(dstp_hbm.at[wid], dst_v)

    def body(j, carry):
        pltpu.sync_copy(ones_v, dacc.at[dst_v.at[j]], add=True)
        return carry

    lax.fori_loop(0, CHUNKS_PER_TILE, body, 0)
    plsc.subcore_barrier()
    pltpu.sync_copy(
        dacc.at[pl.ds(s * ROWS_PER_TILE, ROWS_PER_TILE)],
        deg_hbm.at[c].at[pl.ds(s * ROWS_PER_TILE, ROWS_PER_TILE)],
    )


@functools.partial(
    pl.kernel,
    out_type=jax.ShapeDtypeStruct((NC, N_PAD, D), jnp.float32),
    mesh=_mesh,
    scratch_types=[
        pltpu.VMEM((CHUNKS_PER_TILE, CHUNK), jnp.int32),
        pltpu.VMEM((CHUNKS_PER_TILE, CHUNK), jnp.int32),
        pltpu.VMEM((CHUNK, D), jnp.float32),
        pltpu.VMEM((160, D), jnp.float32),
        pltpu.VMEM_SHARED((N_PAD, D), jnp.float32),
        pltpu.SemaphoreType.DMA,
    ],
)
def _sc_aggregate(g_hbm, srcp_hbm, dstp_hbm, parts_hbm,
                  src_v, dst_v, gbuf, zbuf, acc, sem):
    c = lax.axis_index("c")
    s = lax.axis_index("s")
    wid = s * NC + c

    def fill(i, carry):
        for cg in range(8):
            zbuf[i, pl.ds(cg * 16, 16)] = jnp.zeros((16,), jnp.float32)
        return carry

    lax.fori_loop(0, 160, fill, 0)
    for k in range(ROWS_PER_TILE // 160):
        pltpu.sync_copy(zbuf, acc.at[pl.ds(s * ROWS_PER_TILE + k * 160, 160)])
    plsc.subcore_barrier()

    pltpu.sync_copy(srcp_hbm.at[wid], src_v)
    pltpu.sync_copy(dstp_hbm.at[wid], dst_v)

    def body(j, carry):
        pltpu.async_copy(g_hbm.at[src_v.at[j]], gbuf, sem).wait()
        pltpu.sync_copy(gbuf, acc.at[dst_v.at[j]], add=True)
        return carry

    lax.fori_loop(0, CHUNKS_PER_TILE, body, 0)
    plsc.subcore_barrier()
    pltpu.sync_copy(
        acc.at[pl.ds(s * ROWS_PER_TILE, ROWS_PER_TILE)],
        parts_hbm.at[c].at[pl.ds(s * ROWS_PER_TILE, ROWS_PER_TILE)],
    )


def _tc_scale_body(x_ref, w1_ref, deg_ref, g_ref):
    deg = deg_ref[0, :] + deg_ref[1, :] + 1.0
    dinv = lax.rsqrt(deg)
    h = jnp.dot(x_ref[...], w1_ref[...], preferred_element_type=jnp.float32)
    g_ref[...] = h * dinv[:, None]


_tc_scale = pl.pallas_call(
    _tc_scale_body,
    grid=(N_PAD // 1024,),
    in_specs=[
        pl.BlockSpec((1024, D), lambda i: (i, 0)),
        pl.BlockSpec((D, D), lambda i: (0, 0)),
        pl.BlockSpec((NC, 1024), lambda i: (0, i)),
    ],
    out_specs=pl.BlockSpec((1024, D), lambda i: (i, 0)),
    out_shape=jax.ShapeDtypeStruct((N_PAD, D), jnp.float32),
)


def _tc_head_body(p_ref, g_ref, deg_ref, b1_ref, w2_ref, b2_ref, y_ref):
    deg = deg_ref[0, :] + deg_ref[1, :] + 1.0
    dinv = lax.rsqrt(deg)
    agg = p_ref[0] + p_ref[1] + g_ref[...]
    h = jnp.maximum(agg * dinv[:, None] + b1_ref[...], 0.0)
    y_ref[...] = (
        jnp.dot(h, w2_ref[...], preferred_element_type=jnp.float32)
        + b2_ref[...]
    )


_tc_head = pl.pallas_call(
    _tc_head_body,
    grid=(N_PAD // 1024,),
    in_specs=[
        pl.BlockSpec((NC, 1024, D), lambda i: (0, i, 0)),
        pl.BlockSpec((1024, D), lambda i: (i, 0)),
        pl.BlockSpec((NC, 1024), lambda i: (0, i)),
        pl.BlockSpec((1, D), lambda i: (0, 0)),
        pl.BlockSpec((D, D), lambda i: (0, 0)),
        pl.BlockSpec((1, D), lambda i: (0, 0)),
    ],
    out_specs=pl.BlockSpec((1024, D), lambda i: (i, 0)),
    out_shape=jax.ShapeDtypeStruct((N_PAD, D), jnp.float32),
)


@jax.jit
def kernel(x, edge_index, W1, b1, W2, b2):
    src = edge_index[0]
    dst = edge_index[1]
    pad = E_PAD - E
    srcp = jnp.concatenate([src, jnp.zeros((pad,), jnp.int32)])
    dstp = jnp.concatenate([dst, jnp.full((pad,), DUMMY_DST, jnp.int32)])
    srcp = srcp.reshape(NW, CHUNKS_PER_TILE, CHUNK)
    dstp = dstp.reshape(NW, CHUNKS_PER_TILE, CHUNK)
    x_p = jnp.concatenate([x, jnp.zeros((N_PAD - N, x.shape[1]), x.dtype)])

    deg = _sc_degree(dstp)
    g = _tc_scale(x_p, W1, deg)
    parts = _sc_aggregate(g, srcp, dstp)
    y = _tc_head(parts, g, deg, b1.reshape(1, D), W2, b2.reshape(1, D))
    return y[:N]


# speedup vs baseline: 37.0721x; 37.0721x over previous
import functools

import jax
import jax.numpy as jnp
from jax import lax
from jax.experimental import pallas as pl
from jax.experimental.pallas import tpu as pltpu
from jax.experimental.pallas import tpu_sc as plsc

N = 10000
E = 320000
D = 128
DH = D // 2

NC = 2
NS = 16
NW = NC * NS

N_PAD = 10240
ROWS_PER_TILE = N_PAD // NS
CHUNK = 128
CHUNKS_PER_TILE = 160
E_PAD = NS * CHUNKS_PER_TILE * CHUNK
DUMMY_DST = N

_mesh = plsc.VectorSubcoreMesh(core_axis_name="c", subcore_axis_name="s")


@functools.partial(
    pl.kernel,
    out_type=jax.ShapeDtypeStruct((NC, N_PAD), jnp.float32),
    mesh=_mesh,
    scratch_types=[
        pltpu.VMEM((CHUNKS_PER_TILE // 2, CHUNK), jnp.int32),
        pltpu.VMEM((CHUNK,), jnp.float32),
        pltpu.VMEM((ROWS_PER_TILE,), jnp.float32),
        pltpu.VMEM_SHARED((N_PAD,), jnp.float32),
    ],
)
def _sc_degree(dstp_hbm, deg_hbm, dst_v, ones_v, zbuf, dacc):
    c = lax.axis_index("c")
    s = lax.axis_index("s")
    wid = c * NS + s

    def fill(i, carry):
        zbuf[pl.ds(i * 16, 16)] = jnp.zeros((16,), jnp.float32)
        ones_v[pl.ds(lax.rem(i, 8) * 16, 16)] = jnp.ones((16,), jnp.float32)
        return carry

    lax.fori_loop(0, ROWS_PER_TILE // 16, fill, 0)
    pltpu.sync_copy(zbuf, dacc.at[pl.ds(s * ROWS_PER_TILE, ROWS_PER_TILE)])
    plsc.subcore_barrier()

    pltpu.sync_copy(dstp_hbm.at[wid], dst_v)

    def body(j, carry):
        pltpu.sync_copy(ones_v, dacc.at[dst_v.at[j]], add=True)
        return carry

    lax.fori_loop(0, CHUNKS_PER_TILE // 2, body, 0)
    plsc.subcore_barrier()
    pltpu.sync_copy(
        dacc.at[pl.ds(s * ROWS_PER_TILE, ROWS_PER_TILE)],
        deg_hbm.at[c].at[pl.ds(s * ROWS_PER_TILE, ROWS_PER_TILE)],
    )


NBUF = 5
LEAD = 2
IR = 10
ILEAD = 4


@functools.partial(
    pl.kernel,
    out_type=jax.ShapeDtypeStruct((NC, N_PAD, DH), jnp.float32),
    mesh=_mesh,
    scratch_types=[
        pltpu.VMEM((IR, CHUNK), jnp.int32),
        pltpu.VMEM((IR, CHUNK), jnp.int32),
        pltpu.VMEM((NBUF, CHUNK, DH), jnp.float32),
        pltpu.VMEM_SHARED((N_PAD, DH), jnp.float32),
        pltpu.VMEM_SHARED((N_PAD, DH), jnp.float32),
        [pltpu.SemaphoreType.DMA] * NBUF,
        [pltpu.SemaphoreType.DMA] * NBUF,
        [pltpu.SemaphoreType.DMA] * IR,
        [pltpu.SemaphoreType.DMA] * IR,
    ],
    compiler_params=pltpu.CompilerParams(use_tc_tiling_on_sc=False),
)
def _sc_aggregate(g_hbm, srcp_hbm, dstp_hbm, parts_hbm,
                  idx_s, idx_d, gbuf, acc, gtab, sems_g, sems_s,
                  sems_is, sems_id):
    c = lax.axis_index("c")
    s = lax.axis_index("s")

    def fill(i, carry):
        for cg in range(DH // 16):
            gbuf[0, i, pl.ds(cg * 16, 16)] = jnp.zeros((16,), jnp.float32)
        return carry

    lax.fori_loop(0, CHUNK, fill, 0)
    for k in range(ROWS_PER_TILE // CHUNK):
        pltpu.sync_copy(
            gbuf.at[0], acc.at[pl.ds(s * ROWS_PER_TILE + k * CHUNK, CHUNK)]
        )
    pltpu.sync_copy(
        g_hbm.at[c].at[pl.ds(s * ROWS_PER_TILE, ROWS_PER_TILE)],
        gtab.at[pl.ds(s * ROWS_PER_TILE, ROWS_PER_TILE)],
    )
    plsc.subcore_barrier()

    def idx_cp(j, r):
        pltpu.async_copy(srcp_hbm.at[s].at[j], idx_s.at[r], sems_is[r])
        pltpu.async_copy(dstp_hbm.at[s].at[j], idx_d.at[r], sems_id[r])

    def idx_wait_s(j, r):
        pltpu.make_async_copy(
            srcp_hbm.at[s].at[j], idx_s.at[r], sems_is[r]
        ).wait()

    def idx_wait_d(j, r):
        pltpu.make_async_copy(
            dstp_hbm.at[s].at[j], idx_d.at[r], sems_id[r]
        ).wait()

    def gth(j, b, r):
        pltpu.async_copy(gtab.at[idx_s.at[r]], gbuf.at[b], sems_g[b])

    def gth_wait(j, b, r):
        pltpu.make_async_copy(
            gtab.at[idx_s.at[r]], gbuf.at[b], sems_g[b]
        ).wait()

    def sct(j, b, r):
        pltpu.async_copy(
            gbuf.at[b], acc.at[idx_d.at[r]], sems_s[b], add=True
        )

    def sct_wait(j, b, r):
        pltpu.make_async_copy(
            gbuf.at[b], acc.at[idx_d.at[r]], sems_s[b]
        ).wait()

    for j0 in range(ILEAD):
        idx_cp(j0, j0)
    for j0 in range(LEAD):
        idx_wait_s(j0, j0)
        gth(j0, j0, j0)

    def group(i, carry):
        for u in range(IR):
            j = i * IR + u
            b = u % NBUF
            bp = (u - LEAD) % NBUF
            bn = (u + LEAD) % NBUF
            rn = (u + LEAD) % IR
            ri = (u + ILEAD) % IR

            @pl.when(j >= LEAD)
            def _():
                sct_wait(j - LEAD, bp, (u - LEAD) % IR)

            @pl.when(j + ILEAD < CHUNKS_PER_TILE)
            def _():
                idx_cp(j + ILEAD, ri)

            @pl.when(j + LEAD < CHUNKS_PER_TILE)
            def _():
                idx_wait_s(j + LEAD, rn)
                gth(j + LEAD, bn, rn)

            gth_wait(j, b, u % IR)
            idx_wait_d(j, u % IR)
            sct(j, b, u % IR)
        return carry

    lax.fori_loop(0, CHUNKS_PER_TILE // IR, group, 0)
    for jt in range(CHUNKS_PER_TILE - LEAD, CHUNKS_PER_TILE):
        sct_wait(jt, jt % NBUF, jt % IR)
    plsc.subcore_barrier()
    pltpu.sync_copy(
        acc.at[pl.ds(s * ROWS_PER_TILE, ROWS_PER_TILE)],
        parts_hbm.at[c].at[pl.ds(s * ROWS_PER_TILE, ROWS_PER_TILE)],
    )


def _tc_scale_body(x_ref, w1_ref, deg_ref, g_ref):
    deg = deg_ref[0, :] + deg_ref[1, :] + 1.0
    dinv = lax.rsqrt(deg)
    h = jnp.dot(x_ref[...], w1_ref[...], preferred_element_type=jnp.float32)
    g = h * dinv[:, None]
    g_ref[0, :, :] = g[:, :DH]
    g_ref[1, :, :] = g[:, DH:]


_tc_scale = pl.pallas_call(
    _tc_scale_body,
    grid=(N_PAD // 1024,),
    in_specs=[
        pl.BlockSpec((1024, D), lambda i: (i, 0)),
        pl.BlockSpec((D, D), lambda i: (0, 0)),
        pl.BlockSpec((NC, 1024), lambda i: (0, i)),
    ],
    out_specs=pl.BlockSpec((NC, 1024, DH), lambda i: (0, i, 0)),
    out_shape=jax.ShapeDtypeStruct((NC, N_PAD, DH), jnp.float32),
)


def _tc_head_body(p_ref, g_ref, deg_ref, b1_ref, w2_ref, b2_ref, y_ref):
    deg = deg_ref[0, :] + deg_ref[1, :] + 1.0
    dinv = lax.rsqrt(deg)
    agg = jnp.concatenate(
        [p_ref[0] + g_ref[0], p_ref[1] + g_ref[1]], axis=-1
    )
    h = jnp.maximum(agg * dinv[:, None] + b1_ref[...], 0.0)
    y_ref[...] = (
        jnp.dot(h, w2_ref[...], preferred_element_type=jnp.float32)
        + b2_ref[...]
    )


_tc_head = pl.pallas_call(
    _tc_head_body,
    grid=(N_PAD // 1024,),
    in_specs=[
        pl.BlockSpec((NC, 1024, DH), lambda i: (0, i, 0)),
        pl.BlockSpec((NC, 1024, DH), lambda i: (0, i, 0)),
        pl.BlockSpec((NC, 1024), lambda i: (0, i)),
        pl.BlockSpec((1, D), lambda i: (0, 0)),
        pl.BlockSpec((D, D), lambda i: (0, 0)),
        pl.BlockSpec((1, D), lambda i: (0, 0)),
    ],
    out_specs=pl.BlockSpec((1024, D), lambda i: (i, 0)),
    out_shape=jax.ShapeDtypeStruct((N_PAD, D), jnp.float32),
)


@jax.jit
def kernel(x, edge_index, W1, b1, W2, b2):
    src = edge_index[0]
    dst = edge_index[1]
    pad = E_PAD - E
    srcp = jnp.concatenate([src, jnp.zeros((pad,), jnp.int32)])
    dstp = jnp.concatenate([dst, jnp.full((pad,), DUMMY_DST, jnp.int32)])
    srcp = srcp.reshape(NS, CHUNKS_PER_TILE, CHUNK)
    dstp = dstp.reshape(NS, CHUNKS_PER_TILE, CHUNK)
    dstp_deg = dstp.reshape(NW, CHUNKS_PER_TILE // 2, CHUNK)
    x_p = jnp.concatenate([x, jnp.zeros((N_PAD - N, x.shape[1]), x.dtype)])

    deg = _sc_degree(dstp_deg)
    g = _tc_scale(x_p, W1, deg)
    parts = _sc_aggregate(g, srcp, dstp)
    y = _tc_head(parts, g, deg, b1.reshape(1, D), W2, b2.reshape(1, D))
    return y[:N]


# speedup vs baseline: 37.5397x; 1.0126x over previous
import functools

import jax
import jax.numpy as jnp
from jax import lax
from jax.experimental import pallas as pl
from jax.experimental.pallas import tpu as pltpu
from jax.experimental.pallas import tpu_sc as plsc

N = 10000
E = 320000
D = 128
DH = D // 2

NC = 2
NS = 16
NW = NC * NS

N_PAD = 10240
ROWS_PER_TILE = N_PAD // NS
CHUNK = 128
CHUNKS_PER_TILE = 160
E_PAD = NS * CHUNKS_PER_TILE * CHUNK
DUMMY_DST = N

_mesh = plsc.VectorSubcoreMesh(core_axis_name="c", subcore_axis_name="s")


@functools.partial(
    pl.kernel,
    out_type=jax.ShapeDtypeStruct((NC, N_PAD), jnp.float32),
    mesh=_mesh,
    scratch_types=[
        pltpu.VMEM((CHUNKS_PER_TILE // 2, CHUNK), jnp.int32),
        pltpu.VMEM((CHUNK,), jnp.float32),
        pltpu.VMEM((ROWS_PER_TILE,), jnp.float32),
        pltpu.VMEM_SHARED((N_PAD,), jnp.float32),
    ],
)
def _sc_degree(dstp_hbm, deg_hbm, dst_v, ones_v, zbuf, dacc):
    c = lax.axis_index("c")
    s = lax.axis_index("s")
    wid = c * NS + s

    def fill(i, carry):
        zbuf[pl.ds(i * 16, 16)] = jnp.zeros((16,), jnp.float32)
        ones_v[pl.ds(lax.rem(i, 8) * 16, 16)] = jnp.ones((16,), jnp.float32)
        return carry

    lax.fori_loop(0, ROWS_PER_TILE // 16, fill, 0)
    pltpu.sync_copy(zbuf, dacc.at[pl.ds(s * ROWS_PER_TILE, ROWS_PER_TILE)])
    plsc.subcore_barrier()

    pltpu.sync_copy(dstp_hbm.at[wid], dst_v)

    def body(j, carry):
        pltpu.sync_copy(ones_v, dacc.at[dst_v.at[j]], add=True)
        return carry

    lax.fori_loop(0, CHUNKS_PER_TILE // 2, body, 0)
    plsc.subcore_barrier()
    pltpu.sync_copy(
        dacc.at[pl.ds(s * ROWS_PER_TILE, ROWS_PER_TILE)],
        deg_hbm.at[c].at[pl.ds(s * ROWS_PER_TILE, ROWS_PER_TILE)],
    )


@functools.partial(
    pl.kernel,
    out_type=jax.ShapeDtypeStruct((NC, N_PAD, DH), jnp.float32),
    mesh=_mesh,
    scratch_types=[
        pltpu.VMEM((10, CHUNK), jnp.int32),
        pltpu.VMEM((10, CHUNK), jnp.int32),
        pltpu.VMEM((5, CHUNK, DH), jnp.float32),
        pltpu.VMEM_SHARED((N_PAD, DH), jnp.float32),
        pltpu.VMEM_SHARED((N_PAD, DH), jnp.float32),
        [pltpu.SemaphoreType.DMA] * 5,
        [pltpu.SemaphoreType.DMA] * 5,
        [pltpu.SemaphoreType.DMA] * 10,
        [pltpu.SemaphoreType.DMA] * 10,
    ],
    compiler_params=pltpu.CompilerParams(use_tc_tiling_on_sc=False),
)
def _sc_aggregate(g_hbm, srcp_hbm, dstp_hbm, parts_hbm,
                  idx_s, idx_d, gbuf, acc, gtab, sems_g, sems_s,
                  sems_is, sems_id):
    c = lax.axis_index("c")
    s = lax.axis_index("s")
    NBUF = 5
    LEAD = 2
    IR = 10
    ILEAD = 4

    def fill(i, carry):
        for cg in range(DH // 16):
            gbuf[0, i, pl.ds(cg * 16, 16)] = jnp.zeros((16,), jnp.float32)
        return carry

    lax.fori_loop(0, CHUNK, fill, 0)
    for k in range(ROWS_PER_TILE // CHUNK):
        pltpu.sync_copy(
            gbuf.at[0], acc.at[pl.ds(s * ROWS_PER_TILE + k * CHUNK, CHUNK)]
        )
    pltpu.sync_copy(
        g_hbm.at[c].at[pl.ds(s * ROWS_PER_TILE, ROWS_PER_TILE)],
        gtab.at[pl.ds(s * ROWS_PER_TILE, ROWS_PER_TILE)],
    )
    plsc.subcore_barrier()

    def idx_cp(j, r):
        pltpu.async_copy(srcp_hbm.at[s].at[j], idx_s.at[r], sems_is[r])
        pltpu.async_copy(dstp_hbm.at[s].at[j], idx_d.at[r], sems_id[r])

    def idx_wait_s(j, r):
        pltpu.make_async_copy(
            srcp_hbm.at[s].at[j], idx_s.at[r], sems_is[r]
        ).wait()

    def idx_wait_d(j, r):
        pltpu.make_async_copy(
            dstp_hbm.at[s].at[j], idx_d.at[r], sems_id[r]
        ).wait()

    def gth(j, b, r):
        pltpu.async_copy(gtab.at[idx_s.at[r]], gbuf.at[b], sems_g[b])

    def gth_wait(j, b, r):
        pltpu.make_async_copy(
            gtab.at[idx_s.at[r]], gbuf.at[b], sems_g[b]
        ).wait()

    def sct(j, b, r):
        pltpu.async_copy(
            gbuf.at[b], acc.at[idx_d.at[r]], sems_s[b], add=True
        )

    def sct_wait(j, b, r):
        pltpu.make_async_copy(
            gbuf.at[b], acc.at[idx_d.at[r]], sems_s[b]
        ).wait()

    for j0 in range(ILEAD):
        idx_cp(j0, j0)
    for j0 in range(LEAD):
        idx_wait_s(j0, j0)
        gth(j0, j0, j0)

    def group(i, carry):
        for u in range(IR):
            j = i * IR + u
            b = u % NBUF
            bp = (u - LEAD) % NBUF
            bn = (u + LEAD) % NBUF
            rn = (u + LEAD) % IR
            ri = (u + ILEAD) % IR

            @pl.when(j >= LEAD)
            def _():
                sct_wait(j - LEAD, bp, (u - LEAD) % IR)

            @pl.when(j + ILEAD < CHUNKS_PER_TILE)
            def _():
                idx_cp(j + ILEAD, ri)

            @pl.when(j + LEAD < CHUNKS_PER_TILE)
            def _():
                idx_wait_s(j + LEAD, rn)
                gth(j + LEAD, bn, rn)

            gth_wait(j, b, u % IR)
            idx_wait_d(j, u % IR)
            sct(j, b, u % IR)
        return carry

    lax.fori_loop(0, CHUNKS_PER_TILE // IR, group, 0)
    for jt in range(CHUNKS_PER_TILE - LEAD, CHUNKS_PER_TILE):
        sct_wait(jt, jt % NBUF, jt % IR)
    plsc.subcore_barrier()
    pltpu.sync_copy(
        acc.at[pl.ds(s * ROWS_PER_TILE, ROWS_PER_TILE)],
        parts_hbm.at[c].at[pl.ds(s * ROWS_PER_TILE, ROWS_PER_TILE)],
    )


def _tc_scale_body(x_ref, w1_ref, deg_ref, g_ref, dinv_ref):
    deg = deg_ref[0, :] + deg_ref[1, :] + 1.0
    dinv = lax.rsqrt(deg)
    h = jnp.dot(x_ref[...], w1_ref[...], preferred_element_type=jnp.float32)
    g = h * dinv[:, None]
    g_ref[0, :, :] = g[:, :DH]
    g_ref[1, :, :] = g[:, DH:]
    dinv_ref[...] = dinv[:, None]


_tc_scale = pl.pallas_call(
    _tc_scale_body,
    grid=(N_PAD // 1024,),
    in_specs=[
        pl.BlockSpec((1024, D), lambda i: (i, 0)),
        pl.BlockSpec((D, D), lambda i: (0, 0)),
        pl.BlockSpec((NC, 1024), lambda i: (0, i)),
    ],
    out_specs=[
        pl.BlockSpec((NC, 1024, DH), lambda i: (0, i, 0)),
        pl.BlockSpec((1024, 1), lambda i: (i, 0)),
    ],
    out_shape=[
        jax.ShapeDtypeStruct((NC, N_PAD, DH), jnp.float32),
        jax.ShapeDtypeStruct((N_PAD, 1), jnp.float32),
    ],
)


def _tc_head_body(p_ref, g_ref, dinv_ref, b1_ref, w2_ref, b2_ref, y_ref):
    agg = jnp.concatenate(
        [p_ref[0] + g_ref[0], p_ref[1] + g_ref[1]], axis=-1
    )
    h = jnp.maximum(agg * dinv_ref[...] + b1_ref[...], 0.0)
    y_ref[...] = (
        jnp.dot(h, w2_ref[...], preferred_element_type=jnp.float32)
        + b2_ref[...]
    )


_tc_head = pl.pallas_call(
    _tc_head_body,
    grid=(10,),
    in_specs=[
        pl.BlockSpec((NC, 1000, DH), lambda i: (0, i, 0)),
        pl.BlockSpec((NC, 1000, DH), lambda i: (0, i, 0)),
        pl.BlockSpec((1000, 1), lambda i: (i, 0)),
        pl.BlockSpec((1, D), lambda i: (0, 0)),
        pl.BlockSpec((D, D), lambda i: (0, 0)),
        pl.BlockSpec((1, D), lambda i: (0, 0)),
    ],
    out_specs=pl.BlockSpec((1000, D), lambda i: (i, 0)),
    out_shape=jax.ShapeDtypeStruct((N, D), jnp.float32),
)


@jax.jit
def kernel(x, edge_index, W1, b1, W2, b2):
    src = edge_index[0]
    dst = edge_index[1]
    pad = E_PAD - E
    srcp = jnp.concatenate([src, jnp.zeros((pad,), jnp.int32)])
    dstp = jnp.concatenate([dst, jnp.full((pad,), DUMMY_DST, jnp.int32)])
    srcp = srcp.reshape(NS, CHUNKS_PER_TILE, CHUNK)
    dstp = dstp.reshape(NS, CHUNKS_PER_TILE, CHUNK)
    dstp_deg = dstp.reshape(NW, CHUNKS_PER_TILE // 2, CHUNK)
    deg = _sc_degree(dstp_deg)
    g, dinv = _tc_scale(x, W1, deg)
    parts = _sc_aggregate(g, srcp, dstp)
    return _tc_head(parts, g, dinv, b1.reshape(1, D), W2, b2.reshape(1, D))
